# baseline (speedup 1.0000x reference)
import jax
import jax.numpy as jnp
from jax import lax
from jax.experimental import pallas as pl
from jax.experimental.pallas import tpu as pltpu
from jax.experimental.pallas import tpu_sc as plsc

N_NODES = 100000
TYPES = 100
DIM = 128
NW = 32
CHUNK = 128
SUPER = 2 * CHUNK
NSUP = 12
BIG_W = 20
PER_BIG = NSUP * SUPER + 56
PER_SMALL = NSUP * SUPER + 48
TOFF = NSUP * SUPER
NBUF = 3


def _body(idx_hbm, table_hbm, out_hbm, table_v, idx_v,
          b0, b1, b2, gsem, s0, s1, s2):
    wid = lax.axis_index("s") * 2 + lax.axis_index("c")
    bufs = (b0, b1, b2)
    sems = (s0, s1, s2)

    base = PER_SMALL * wid + 8 * jnp.minimum(wid, BIG_W)
    is_big = wid < BIG_W

    @pl.when(lax.axis_index("s") == 0)
    def _stage_table():
        pltpu.async_copy(table_hbm, table_v, gsem)

    @pl.when(is_big)
    def _stage_idx_big():
        pltpu.sync_copy(idx_hbm.at[pl.ds(base, PER_BIG)],
                        idx_v.at[pl.ds(0, PER_BIG)])

    @pl.when(jnp.logical_not(is_big))
    def _stage_idx_small():
        pltpu.sync_copy(idx_hbm.at[pl.ds(base, PER_SMALL)],
                        idx_v.at[pl.ds(0, PER_SMALL)])

    @pl.when(lax.axis_index("s") == 0)
    def _wait_table():
        pltpu.make_async_copy(table_hbm, table_v, gsem).wait()
    plsc.subcore_barrier()

    def gather_issue(k, b):
        for h in range(2):
            pltpu.async_copy(
                table_v.at[idx_v.at[pl.ds(k * SUPER + h * CHUNK, CHUNK)]],
                bufs[b].at[pl.ds(h * CHUNK, CHUNK)], gsem)

    def gather_wait(k, b):
        for h in range(2):
            pltpu.make_async_copy(
                table_v.at[idx_v.at[pl.ds(k * SUPER + h * CHUNK, CHUNK)]],
                bufs[b].at[pl.ds(h * CHUNK, CHUNK)], gsem).wait()

    def scat_issue(k, b):
        pltpu.async_copy(
            bufs[b], out_hbm.at[pl.ds(base + k * SUPER, SUPER)], sems[b])

    def scat_wait(k, b):
        pltpu.make_async_copy(
            bufs[b], out_hbm.at[pl.ds(base + k * SUPER, SUPER)], sems[b]).wait()

    def tail_both(op):
        @pl.when(is_big)
        def _t_big():
            op(56)
        @pl.when(jnp.logical_not(is_big))
        def _t_small():
            op(48)

    def tail_gather_issue():
        tail_both(lambda n: pltpu.async_copy(
            table_v.at[idx_v.at[pl.ds(TOFF, n)]], b0.at[pl.ds(0, n)], gsem))

    def tail_gather_wait():
        tail_both(lambda n: pltpu.make_async_copy(
            table_v.at[idx_v.at[pl.ds(TOFF, n)]], b0.at[pl.ds(0, n)],
            gsem).wait())

    def tail_scat_issue():
        tail_both(lambda n: pltpu.async_copy(
            b0.at[pl.ds(0, n)], out_hbm.at[pl.ds(base + TOFF, n)], s0))

    def tail_scat_wait():
        tail_both(lambda n: pltpu.make_async_copy(
            b0.at[pl.ds(0, n)], out_hbm.at[pl.ds(base + TOFF, n)], s0).wait())

    gather_issue(0, 0)
    for k in (0, 1):
        gather_issue(k + 1, k + 1)
        gather_wait(k, k)
        scat_issue(k, k)

    def steady(i, carry):
        for t in range(NBUF):
            k = i * NBUF + t - 1
            b = (t + 2) % NBUF
            nb = t % NBUF
            scat_wait(k - 2, nb)
            gather_issue(k + 1, nb)
            gather_wait(k, b)
            scat_issue(k, b)
        return carry

    lax.fori_loop(1, 4, steady, 0)

    scat_wait(9, 0)
    tail_gather_issue()
    gather_wait(11, 2)
    scat_issue(11, 2)

    tail_gather_wait()
    tail_scat_issue()

    scat_wait(10, 1)
    scat_wait(11, 2)
    tail_scat_wait()


def kernel(node_type, table):
    mesh = plsc.VectorSubcoreMesh(core_axis_name="c", subcore_axis_name="s")
    f = pl.kernel(
        _body,
        mesh=mesh,
        out_type=jax.ShapeDtypeStruct((N_NODES, DIM), jnp.float32),
        scratch_types=[
            pltpu.VMEM_SHARED((TYPES, DIM), jnp.float32),
            pltpu.VMEM((PER_BIG,), jnp.int32),
            *[pltpu.VMEM((SUPER, DIM), jnp.float32) for _ in range(NBUF)],
            pltpu.SemaphoreType.DMA,
            *[pltpu.SemaphoreType.DMA for _ in range(NBUF)],
        ],
    )
    return f(node_type.astype(jnp.int32), table)


# speedup vs baseline: 1.0140x; 1.0140x over previous
import jax
import jax.numpy as jnp
from jax import lax
from jax.experimental import pallas as pl
from jax.experimental.pallas import tpu as pltpu
from jax.experimental.pallas import tpu_sc as plsc

N_NODES = 100000
TYPES = 100
DIM = 128
NW = 32
CHUNK = 128
FULL = 24
BIG_W = 20
PER_BIG = FULL * CHUNK + 56
PER_SMALL = FULL * CHUNK + 48
TOFF = FULL * CHUNK
NBUF = 4


def _body(idx_hbm, table_hbm, out_hbm, table_v, idx_v,
          b0, b1, b2, b3, gsem, s0, s1, s2, s3):
    wid = lax.axis_index("s") * 2 + lax.axis_index("c")
    bufs = (b0, b1, b2, b3)
    sems = (s0, s1, s2, s3)

    base = PER_SMALL * wid + 8 * jnp.minimum(wid, BIG_W)
    is_big = wid < BIG_W

    @pl.when(lax.axis_index("s") == 0)
    def _stage_table():
        pltpu.async_copy(table_hbm, table_v, gsem)

    @pl.when(is_big)
    def _stage_idx_big():
        pltpu.sync_copy(idx_hbm.at[pl.ds(base, PER_BIG)],
                        idx_v.at[pl.ds(0, PER_BIG)])

    @pl.when(jnp.logical_not(is_big))
    def _stage_idx_small():
        pltpu.sync_copy(idx_hbm.at[pl.ds(base, PER_SMALL)],
                        idx_v.at[pl.ds(0, PER_SMALL)])

    @pl.when(lax.axis_index("s") == 0)
    def _wait_table():
        pltpu.make_async_copy(table_hbm, table_v, gsem).wait()
    plsc.subcore_barrier()

    def out_base(j):
        return base + j * CHUNK

    def gather_issue(j, b):
        pltpu.async_copy(
            table_v.at[idx_v.at[pl.ds(j * CHUNK, CHUNK)]], bufs[b], gsem)

    def gather_wait(j, b):
        pltpu.make_async_copy(
            table_v.at[idx_v.at[pl.ds(j * CHUNK, CHUNK)]], bufs[b], gsem).wait()

    def scat_issue(j, b):
        pltpu.async_copy(bufs[b], out_hbm.at[pl.ds(out_base(j), CHUNK)], sems[b])

    def scat_wait(j, b):
        pltpu.make_async_copy(
            bufs[b], out_hbm.at[pl.ds(out_base(j), CHUNK)], sems[b]).wait()

    def tail_both(op):
        @pl.when(is_big)
        def _t_big():
            op(56)
        @pl.when(jnp.logical_not(is_big))
        def _t_small():
            op(48)

    def tail_gather_issue():
        tail_both(lambda n: pltpu.async_copy(
            table_v.at[idx_v.at[pl.ds(TOFF, n)]], b0.at[pl.ds(0, n)], gsem))

    def tail_gather_wait():
        tail_both(lambda n: pltpu.make_async_copy(
            table_v.at[idx_v.at[pl.ds(TOFF, n)]], b0.at[pl.ds(0, n)],
            gsem).wait())

    def tail_scat_issue():
        tail_both(lambda n: pltpu.async_copy(
            b0.at[pl.ds(0, n)], out_hbm.at[pl.ds(base + TOFF, n)], s0))

    def tail_scat_wait():
        tail_both(lambda n: pltpu.make_async_copy(
            b0.at[pl.ds(0, n)], out_hbm.at[pl.ds(base + TOFF, n)], s0).wait())

    gather_issue(0, 0)
    for j in range(3):
        gather_issue(j + 1, j + 1)
        gather_wait(j, j)
        scat_issue(j, j)

    scat_wait(0, 0)
    gather_issue(4, 0)
    gather_wait(3, 3)
    scat_issue(3, 3)

    def steady(i, carry):
        for b in range(NBUF):
            j = i * NBUF + b
            nb = (b + 1) % NBUF
            scat_wait(j - 3, nb)
            gather_issue(j + 1, nb)
            gather_wait(j, b)
            scat_issue(j, b)
        return carry

    lax.fori_loop(1, FULL // NBUF - 1, steady, 0)

    for j in range(20, 24):
        b = j % NBUF
        nb = (b + 1) % NBUF
        scat_wait(j - 3, nb)
        if j < 23:
            gather_issue(j + 1, nb)
        else:
            tail_gather_issue()
        gather_wait(j, b)
        scat_issue(j, b)

    tail_gather_wait()
    tail_scat_issue()

    scat_wait(21, 1)
    scat_wait(22, 2)
    scat_wait(23, 3)
    tail_scat_wait()


def kernel(node_type, table):
    mesh = plsc.VectorSubcoreMesh(core_axis_name="c", subcore_axis_name="s")
    f = pl.kernel(
        _body,
        mesh=mesh,
        out_type=jax.ShapeDtypeStruct((N_NODES, DIM), jnp.float32),
        scratch_types=[
            pltpu.VMEM_SHARED((TYPES, DIM), jnp.float32),
            pltpu.VMEM((PER_BIG,), jnp.int32),
            *[pltpu.VMEM((CHUNK, DIM), jnp.float32) for _ in range(NBUF)],
            pltpu.SemaphoreType.DMA,
            *[pltpu.SemaphoreType.DMA for _ in range(NBUF)],
        ],
    )
    return f(node_type.astype(jnp.int32), table)
